# baseline (speedup 1.0000x reference)
import jax
import jax.numpy as jnp
from jax import lax
from jax.experimental import pallas as pl
from jax.experimental.pallas import tpu as pltpu
from jax.experimental.pallas import tpu_sc as plsc

NSEG = 256
D = 128
NWORK = 32
CHUNK = 400
BLK = 6400
T2 = 128
NT2 = BLK // T2


def _sc_stats_body(x_hbm, ids_hbm, psum_hbm, psq_hbm, pcnt_hbm,
                   xbuf, idbuf, sacc, qacc, cacc):
    rpw = x_hbm.shape[0] // (D * NWORK)
    nchunk = rpw // CHUNK
    wid = lax.axis_index("s") * 2 + lax.axis_index("c")
    base0 = wid * rpw

    z16 = jnp.zeros((16,), jnp.float32)

    def zbody(g, _):
        sacc[pl.ds(g * 16, 16)] = z16
        qacc[pl.ds(g * 16, 16)] = z16
        return 0

    lax.fori_loop(0, (NSEG * D) // 16, zbody, 0)

    def zcbody(g, _):
        cacc[pl.ds(g * 16, 16)] = z16
        return 0

    lax.fori_loop(0, NSEG, zcbody, 0)

    def chunk_body(ci, _):
        rbase = base0 + ci * CHUNK
        pltpu.sync_copy(x_hbm.at[pl.ds(rbase * D, CHUNK * D)], xbuf)
        pltpu.sync_copy(ids_hbm.at[pl.ds(rbase, CHUNK)], idbuf)
        first = idbuf[pl.ds(0, 16)][0]
        last = idbuf[pl.ds(CHUNK - 16, 16)][15]

        def seg_body(s, lo):
            def grp_body(g, acc):
                head = idbuf[pl.ds(g * 16, 16)][0]
                return acc + jnp.where(head <= s, 1, 0).astype(jnp.int32)

            ng = lax.fori_loop(0, CHUNK // 16, grp_body, jnp.int32(0))
            gb = jnp.maximum(ng - 1, 0)
            vg = idbuf[pl.ds(gb * 16, 16)]
            cnt_in = jnp.int32(0)
            for lane in range(16):
                cnt_in = cnt_in + jnp.where(
                    vg[lane] <= s, 1, 0).astype(jnp.int32)
            hi = 16 * gb + cnt_in

            def row_body(r, accs):
                out = list(accs)
                for j in range(8):
                    v = xbuf[pl.ds(r * D + j * 16, 16)]
                    out[j] = accs[j] + v
                    out[8 + j] = accs[8 + j] + v * v
                return tuple(out)

            accs = lax.fori_loop(lo, hi, row_body,
                                 tuple(z16 for _ in range(16)))
            for j in range(8):
                sacc[pl.ds(s * D + j * 16, 16)] += accs[j]
                qacc[pl.ds(s * D + j * 16, 16)] += accs[8 + j]
            cacc[pl.ds(s * 16, 16)] += (
                jnp.full((16,), 1.0) * (hi - lo).astype(jnp.float32))
            return hi

        lax.fori_loop(first, last + 1, seg_body, jnp.int32(0))
        return 0

    lax.fori_loop(0, nchunk, chunk_body, 0)

    pltpu.sync_copy(sacc, psum_hbm.at[wid])
    pltpu.sync_copy(qacc, psq_hbm.at[wid])
    pltpu.sync_copy(cacc, pcnt_hbm.at[wid])


def _sc_stats(x_flat, ids32):
    mesh = plsc.VectorSubcoreMesh(core_axis_name="c", subcore_axis_name="s")
    f32 = jnp.float32
    return pl.kernel(
        _sc_stats_body,
        out_type=[
            jax.ShapeDtypeStruct((NWORK, NSEG * D), f32),
            jax.ShapeDtypeStruct((NWORK, NSEG * D), f32),
            jax.ShapeDtypeStruct((NWORK, NSEG * 16), f32),
        ],
        scratch_types=[
            pltpu.VMEM((CHUNK * D,), f32),
            pltpu.VMEM((CHUNK,), jnp.int32),
            pltpu.VMEM((NSEG * D,), f32),
            pltpu.VMEM((NSEG * D,), f32),
            pltpu.VMEM((NSEG * 16,), f32),
        ],
        mesh=mesh,
    )(x_flat, ids32)


def _norm_body(x_ref, ids_ref, sum_ref, sq_ref, cnt_ref, w_ref, b_ref,
               o_ref, scale_ref, shift_ref):
    i = pl.program_id(0)

    @pl.when(i == 0)
    def _():
        sums = jnp.sum(sum_ref[...], axis=0)
        sq = jnp.sum(sq_ref[...], axis=0)
        cnt = jnp.maximum(jnp.sum(cnt_ref[...], axis=0)[:, :1], 1.0)
        mean = sums / cnt
        var = sq / cnt - mean * mean
        inv = lax.rsqrt(var + 1e-8)
        w = w_ref[...]
        scale_ref[...] = inv * w
        shift_ref[...] = b_ref[...] - mean * inv * w

    ids = ids_ref[0]
    first = jnp.min(ids)
    last = jnp.max(ids)
    iota = lax.broadcasted_iota(jnp.int32, (T2, 1), 0)

    def seg_body(s, lo):
        hi = jnp.sum((ids <= s).astype(jnp.int32))
        sv = scale_ref[pl.ds(s, 1), :]
        tv = shift_ref[pl.ds(s, 1), :]
        ta = (lo + T2 - 1) // T2
        tb_u = hi // T2
        tb = jnp.minimum(tb_u, NT2 - 1)
        t_a = lo // T2

        nin = jnp.maximum(tb_u - ta, 0)
        npairs = nin // 2

        def tile_body(p, _):
            base = T2 * (ta + 2 * p)
            v1 = x_ref[pl.ds(base, T2), :]
            o_ref[pl.ds(base, T2), :] = v1 * sv + tv
            v2 = x_ref[pl.ds(base + T2, T2), :]
            o_ref[pl.ds(base + T2, T2), :] = v2 * sv + tv
            return 0

        lax.fori_loop(0, npairs, tile_body, 0)

        @pl.when(nin - 2 * npairs == 1)
        def _():
            t_r = ta + 2 * npairs
            v_r = x_ref[pl.ds(T2 * t_r, T2), :]
            o_ref[pl.ds(T2 * t_r, T2), :] = v_r * sv + tv

        v_a = x_ref[pl.ds(T2 * t_a, T2), :]
        r_a = iota + T2 * t_a
        m_a = (r_a >= lo) & (r_a < jnp.minimum(hi, T2 * ta))
        old_a = o_ref[pl.ds(T2 * t_a, T2), :]
        o_ref[pl.ds(T2 * t_a, T2), :] = jnp.where(m_a, v_a * sv + tv, old_a)
        v_b = x_ref[pl.ds(T2 * tb, T2), :]
        r_b = iota + T2 * tb
        m_b = (r_b >= jnp.maximum(lo, T2 * tb_u)) & (r_b < hi) & (tb_u >= ta)
        old_b = o_ref[pl.ds(T2 * tb, T2), :]
        o_ref[pl.ds(T2 * tb, T2), :] = jnp.where(m_b, v_b * sv + tv, old_b)
        return hi

    lax.fori_loop(first, last + 1, seg_body, jnp.int32(0))


def kernel(in_feat, segment_ids, weight, bias):
    n, d = in_feat.shape
    nblk = n // BLK
    ids32 = segment_ids.astype(jnp.int32)
    ids = ids32.reshape(nblk, 8, BLK // 8)

    psum, psq, pcnt = _sc_stats(in_feat.reshape(-1), ids32)
    psum = psum.reshape(NWORK, NSEG, d)
    psq = psq.reshape(NWORK, NSEG, d)
    pcnt = pcnt.reshape(NWORK, NSEG, 16)

    out = pl.pallas_call(
        _norm_body,
        grid=(nblk,),
        in_specs=[
            pl.BlockSpec((BLK, d), lambda i: (i, 0)),
            pl.BlockSpec((1, 8, BLK // 8), lambda i: (i, 0, 0)),
            pl.BlockSpec((NWORK, NSEG, d), lambda i: (0, 0, 0)),
            pl.BlockSpec((NWORK, NSEG, d), lambda i: (0, 0, 0)),
            pl.BlockSpec((NWORK, NSEG, 16), lambda i: (0, 0, 0)),
            pl.BlockSpec((1, d), lambda i: (0, 0)),
            pl.BlockSpec((1, d), lambda i: (0, 0)),
        ],
        out_specs=pl.BlockSpec((BLK, d), lambda i: (i, 0)),
        out_shape=jax.ShapeDtypeStruct((n, d), jnp.float32),
        scratch_shapes=[
            pltpu.VMEM((NSEG, d), jnp.float32),
            pltpu.VMEM((NSEG, d), jnp.float32),
        ],
    )(in_feat, ids, psum, psq, pcnt, weight, bias)
    return out
